# Initial kernel scaffold; baseline (speedup 1.0000x reference)
import jax
import jax.numpy as jnp
from jax.experimental import pallas as pl


def kernel(x, point_cloud, point2frameidx, frame2batchidx, W1, b1, g1, be1, W2, b2, g2, be2, Wvw, bvw, Wvo, bvo, Wyaw, byaw, Wvel, bvel, Wbin, bbin, Wres, bres):
    raise NotImplementedError("write your pallas kernel here")


# speedup vs baseline: 3.2240x; 3.2240x over previous
import jax
import jax.numpy as jnp
import numpy as np
from jax.experimental import pallas as pl
from jax.experimental.pallas import tpu as pltpu

F = 128
N = 262144
BT = 80
B = 8
NBINS = 4
NG = 16
NBLK = 4096
NGRID = N // NBLK
EPS = 1e-5
CNT = float((F // NG) * N)


def _stage1_kernel(x_ref, w_ref, b_ref, h_ref, st_ref):
    i = pl.program_id(0)

    @pl.when(i == 0)
    def _():
        st_ref[...] = jnp.zeros_like(st_ref)

    h = jax.lax.dot_general(x_ref[...], w_ref[...], (((0,), (0,)), ((), ())),
                            preferred_element_type=jnp.float32) + b_ref[...]
    h_ref[...] = h
    st_ref[0:1, :] += jnp.sum(h, axis=0, keepdims=True)
    st_ref[1:2, :] += jnp.sum(h * h, axis=0, keepdims=True)


def _affine(st, g, be, gm):
    sg = jnp.dot(st[0:1, :], gm, preferred_element_type=jnp.float32)
    sgg = jnp.dot(st[1:2, :], gm, preferred_element_type=jnp.float32)
    mu = sg / CNT
    var = sgg / CNT - mu * mu
    a = g * jax.lax.rsqrt(var + EPS)
    d = be - mu * a
    return a, d


def _stage2_kernel(h1_ref, st_ref, g_ref, be_ref, gm_ref, w_ref, b_ref,
                   h2_ref, st2_ref):
    i = pl.program_id(0)

    @pl.when(i == 0)
    def _():
        st2_ref[...] = jnp.zeros_like(st2_ref)

    a, d = _affine(st_ref[...], g_ref[...], be_ref[...], gm_ref[...])
    h1r = jnp.maximum(h1_ref[...] * a + d, 0.0)
    h2 = jax.lax.dot_general(h1r, w_ref[...], (((1,), (0,)), ((), ())),
                             preferred_element_type=jnp.float32) + b_ref[...]
    h2_ref[...] = h2
    st2_ref[0:1, :] += jnp.sum(h2, axis=0, keepdims=True)
    st2_ref[1:2, :] += jnp.sum(h2 * h2, axis=0, keepdims=True)


def _stage3_kernel(bounds_ref, h2_ref, ids_ref, pc_ref, st_ref, g_ref, be_ref,
                   gm_ref, wv_ref, bv_ref, fp_ref, cs_ref):
    i = pl.program_id(0)

    @pl.when(i == 0)
    def _():
        fp_ref[...] = jnp.full_like(fp_ref, -jnp.inf)
        cs_ref[...] = jnp.zeros_like(cs_ref)

    a, d = _affine(st_ref[...], g_ref[...], be_ref[...], gm_ref[...])
    xt = jnp.maximum(h2_ref[...] * a + d, 0.0)
    votes = jax.lax.dot_general(xt, wv_ref[...], (((1,), (0,)), ((), ())),
                                preferred_element_type=jnp.float32) + bv_ref[...]
    vw = jnp.maximum(jax.nn.sigmoid(votes[:, 0:1]), 1e-5)
    cv = (pc_ref[:, 0:3] + votes[:, 1:4]) * vw
    cvw = jnp.concatenate(
        [cv, vw, jnp.zeros((NBLK, 4), jnp.float32)], axis=1)
    ids = ids_ref[...]
    mt = (ids == jax.lax.broadcasted_iota(jnp.int32, (NBLK, BT), 1)
          ).astype(jnp.float32)
    cs_ref[...] += jax.lax.dot_general(mt, cvw, (((0,), (0,)), ((), ())),
                                       preferred_element_type=jnp.float32)

    f0 = bounds_ref[i, 0]
    f1 = bounds_ref[i, 1]

    def body(f, carry):
        m = (ids == f)
        loc = jnp.max(jnp.where(m, xt, -jnp.inf), axis=0, keepdims=True)
        fp_ref[pl.ds(f, 1), :] = jnp.maximum(fp_ref[pl.ds(f, 1), :], loc)
        return carry

    jax.lax.fori_loop(f0, f1 + 1, body, 0)


def _head_kernel(fp_ref, cs_ref, f2b_ref, w5_ref, b5_ref, wsp_ref, bsp_ref,
                 out_ref):
    fp = fp_ref[...]
    f2b = f2b_ref[...]
    rows = []
    for b in range(B):
        m = (f2b == b)
        rows.append(jnp.max(jnp.where(m, fp, -jnp.inf), axis=0, keepdims=True))
    sp = jnp.concatenate(rows, axis=0)
    out5 = jax.lax.dot_general(fp, w5_ref[...], (((1,), (0,)), ((), ())),
                               preferred_element_type=jnp.float32) + b5_ref[...]
    so = jax.lax.dot_general(sp, wsp_ref[...], (((1,), (0,)), ((), ())),
                             preferred_element_type=jnp.float32) + bsp_ref[...]
    res = so[:, 0:12]
    bl = so[:, 12:16]
    mx = jnp.max(bl, axis=1, keepdims=True)
    e = jnp.exp(bl - mx)
    sb = e / jnp.sum(e, axis=1, keepdims=True)
    sb = jnp.where(jnp.isfinite(sb), sb, 0.0)
    res = jnp.where(jnp.isfinite(res), res, 0.0)
    mb = (f2b == jax.lax.broadcasted_iota(jnp.int32, (BT, B), 1)
          ).astype(jnp.float32)
    bins80 = jnp.dot(mb, sb, preferred_element_type=jnp.float32)
    res80 = jnp.dot(mb, res, preferred_element_type=jnp.float32)
    bmx = jnp.max(bins80, axis=1, keepdims=True)
    i4 = jax.lax.broadcasted_iota(jnp.int32, (BT, NBINS), 1)
    win = jnp.min(jnp.where(bins80 >= bmx, i4, NBINS), axis=1, keepdims=True)
    anchor = jnp.zeros((BT, 3), jnp.float32)
    for k in range(NBINS):
        anchor = anchor + jnp.where(win == k, 1.0, 0.0) * res80[:, 3 * k:3 * k + 3]
    centers = cs_ref[:, 0:3] / cs_ref[:, 3:4]
    vel = out5[:, 0:3]
    yaw = out5[:, 3:5]
    out_ref[...] = jnp.concatenate(
        [centers, anchor, yaw, vel, jnp.zeros((BT, 3), jnp.float32), bins80],
        axis=1)


_GM = np.kron(np.eye(NG, dtype=np.float32), np.ones((F // NG, F // NG), np.float32))


def kernel(x, point_cloud, point2frameidx, frame2batchidx, W1, b1, g1, be1,
           W2, b2, g2, be2, Wvw, bvw, Wvo, bvo, Wyaw, byaw, Wvel, bvel,
           Wbin, bbin, Wres, bres):
    f32 = jnp.float32
    gm = jnp.asarray(_GM)
    ids = point2frameidx.astype(jnp.int32)
    ids_col = ids.reshape(N, 1)
    idsr = ids.reshape(NGRID, NBLK)
    bounds = jnp.stack([idsr[:, 0], idsr[:, -1]], axis=1)

    h1, st1 = pl.pallas_call(
        _stage1_kernel,
        grid=(NGRID,),
        in_specs=[pl.BlockSpec((F, NBLK), lambda i: (0, i)),
                  pl.BlockSpec((F, F), lambda i: (0, 0)),
                  pl.BlockSpec((1, F), lambda i: (0, 0))],
        out_specs=[pl.BlockSpec((NBLK, F), lambda i: (i, 0)),
                   pl.BlockSpec((8, F), lambda i: (0, 0))],
        out_shape=[jax.ShapeDtypeStruct((N, F), f32),
                   jax.ShapeDtypeStruct((8, F), f32)],
    )(x, W1.T, b1.reshape(1, F))

    h2, st2 = pl.pallas_call(
        _stage2_kernel,
        grid=(NGRID,),
        in_specs=[pl.BlockSpec((NBLK, F), lambda i: (i, 0)),
                  pl.BlockSpec((8, F), lambda i: (0, 0)),
                  pl.BlockSpec((1, F), lambda i: (0, 0)),
                  pl.BlockSpec((1, F), lambda i: (0, 0)),
                  pl.BlockSpec((F, F), lambda i: (0, 0)),
                  pl.BlockSpec((F, F), lambda i: (0, 0)),
                  pl.BlockSpec((1, F), lambda i: (0, 0))],
        out_specs=[pl.BlockSpec((NBLK, F), lambda i: (i, 0)),
                   pl.BlockSpec((8, F), lambda i: (0, 0))],
        out_shape=[jax.ShapeDtypeStruct((N, F), f32),
                   jax.ShapeDtypeStruct((8, F), f32)],
    )(h1, st1, g1.reshape(1, F), be1.reshape(1, F), gm, W2.T, b2.reshape(1, F))

    wv8 = jnp.concatenate([Wvw, Wvo, jnp.zeros((4, F), f32)], axis=0).T
    bv8 = jnp.concatenate([bvw, bvo, jnp.zeros((4,), f32)]).reshape(1, 8)

    fp, cs = pl.pallas_call(
        _stage3_kernel,
        grid_spec=pltpu.PrefetchScalarGridSpec(
            num_scalar_prefetch=1,
            grid=(NGRID,),
            in_specs=[pl.BlockSpec((NBLK, F), lambda i, b: (i, 0)),
                      pl.BlockSpec((NBLK, 1), lambda i, b: (i, 0)),
                      pl.BlockSpec((NBLK, 4), lambda i, b: (i, 0)),
                      pl.BlockSpec((8, F), lambda i, b: (0, 0)),
                      pl.BlockSpec((1, F), lambda i, b: (0, 0)),
                      pl.BlockSpec((1, F), lambda i, b: (0, 0)),
                      pl.BlockSpec((F, F), lambda i, b: (0, 0)),
                      pl.BlockSpec((F, 8), lambda i, b: (0, 0)),
                      pl.BlockSpec((1, 8), lambda i, b: (0, 0))],
            out_specs=[pl.BlockSpec((BT, F), lambda i, b: (0, 0)),
                       pl.BlockSpec((BT, 8), lambda i, b: (0, 0))],
        ),
        out_shape=[jax.ShapeDtypeStruct((BT, F), f32),
                   jax.ShapeDtypeStruct((BT, 8), f32)],
    )(bounds, h2, ids_col, point_cloud, st2, g2.reshape(1, F),
      be2.reshape(1, F), gm, wv8, bv8)

    w5 = jnp.concatenate([Wvel, Wyaw, jnp.zeros((3, F), f32)], axis=0).T
    b5 = jnp.concatenate([bvel, byaw, jnp.zeros((3,), f32)]).reshape(1, 8)
    wsp = jnp.concatenate([Wres, Wbin], axis=0).T
    bsp = jnp.concatenate([bres, bbin]).reshape(1, 16)
    f2b = frame2batchidx.astype(jnp.int32).reshape(BT, 1)

    out = pl.pallas_call(
        _head_kernel,
        out_shape=jax.ShapeDtypeStruct((BT, 18), f32),
    )(fp, cs, f2b, w5, b5, wsp, bsp)
    return out


# speedup vs baseline: 3.4032x; 1.0556x over previous
import jax
import jax.numpy as jnp
import numpy as np
from jax.experimental import pallas as pl
from jax.experimental.pallas import tpu as pltpu

F = 128
N = 262144
BT = 80
B = 8
NBINS = 4
NG = 16
NBLK = 4096
NGRID = N // NBLK
EPS = 1e-5
CNT = float((F // NG) * N)


def _stage1_kernel(x_ref, w_ref, b_ref, h_ref, st_ref):
    i = pl.program_id(0)

    @pl.when(i == 0)
    def _():
        st_ref[...] = jnp.zeros_like(st_ref)

    h = jax.lax.dot_general(x_ref[...], w_ref[...], (((0,), (0,)), ((), ())),
                            preferred_element_type=jnp.float32) + b_ref[...]
    h_ref[...] = h.astype(jnp.bfloat16)
    st_ref[0:1, :] += jnp.sum(h, axis=0, keepdims=True)
    st_ref[1:2, :] += jnp.sum(h * h, axis=0, keepdims=True)


def _affine(st, g, be, gm):
    sg = jnp.dot(st[0:1, :], gm, preferred_element_type=jnp.float32)
    sgg = jnp.dot(st[1:2, :], gm, preferred_element_type=jnp.float32)
    mu = sg / CNT
    var = sgg / CNT - mu * mu
    a = g * jax.lax.rsqrt(var + EPS)
    d = be - mu * a
    return a, d


def _stage2_kernel(h1_ref, st_ref, g_ref, be_ref, gm_ref, w_ref, b_ref,
                   h2_ref, st2_ref):
    i = pl.program_id(0)

    @pl.when(i == 0)
    def _():
        st2_ref[...] = jnp.zeros_like(st2_ref)

    a, d = _affine(st_ref[...], g_ref[...], be_ref[...], gm_ref[...])
    h1r = jnp.maximum(h1_ref[...].astype(jnp.float32) * a + d, 0.0)
    h2 = jax.lax.dot_general(h1r, w_ref[...], (((1,), (0,)), ((), ())),
                             preferred_element_type=jnp.float32) + b_ref[...]
    h2_ref[...] = h2.astype(jnp.bfloat16)
    st2_ref[0:1, :] += jnp.sum(h2, axis=0, keepdims=True)
    st2_ref[1:2, :] += jnp.sum(h2 * h2, axis=0, keepdims=True)


def _stage3_kernel(bounds_ref, h2_ref, ids_ref, pc_ref, st_ref, g_ref, be_ref,
                   gm_ref, wv_ref, bv_ref, f2b_ref, w5_ref, b5_ref, wsp_ref,
                   bsp_ref, out_ref, fp_ref, cs_ref):
    i = pl.program_id(0)

    @pl.when(i == 0)
    def _():
        fp_ref[...] = jnp.full_like(fp_ref, -jnp.inf)
        cs_ref[...] = jnp.zeros_like(cs_ref)

    a, d = _affine(st_ref[...], g_ref[...], be_ref[...], gm_ref[...])
    xt = jnp.maximum(h2_ref[...].astype(jnp.float32) * a + d, 0.0)
    votes = jax.lax.dot_general(xt, wv_ref[...], (((1,), (0,)), ((), ())),
                                preferred_element_type=jnp.float32) + bv_ref[...]
    vw = jnp.maximum(jax.nn.sigmoid(votes[:, 0:1]), 1e-5)
    cv = (pc_ref[:, 0:3] + votes[:, 1:4]) * vw
    cvw = jnp.concatenate(
        [cv, vw, jnp.zeros((NBLK, 4), jnp.float32)], axis=1)
    ids = ids_ref[...]
    mt = (ids == jax.lax.broadcasted_iota(jnp.int32, (NBLK, BT), 1)
          ).astype(jnp.float32)
    cs_ref[...] += jax.lax.dot_general(mt, cvw, (((0,), (0,)), ((), ())),
                                       preferred_element_type=jnp.float32)

    f0 = bounds_ref[i, 0]
    f1 = bounds_ref[i, 1]

    def body(f, carry):
        m = (ids == f)
        loc = jnp.max(jnp.where(m, xt, -jnp.inf), axis=0, keepdims=True)
        fp_ref[pl.ds(f, 1), :] = jnp.maximum(fp_ref[pl.ds(f, 1), :], loc)
        return carry

    jax.lax.fori_loop(f0, f1 + 1, body, 0)

    @pl.when(i == NGRID - 1)
    def _():
        fp = fp_ref[...]
        f2b = f2b_ref[...]
        rows = []
        for b in range(B):
            m = (f2b == b)
            rows.append(jnp.max(jnp.where(m, fp, -jnp.inf), axis=0,
                                keepdims=True))
        sp = jnp.concatenate(rows, axis=0)
        out5 = jax.lax.dot_general(fp, w5_ref[...], (((1,), (0,)), ((), ())),
                                   preferred_element_type=jnp.float32
                                   ) + b5_ref[...]
        so = jax.lax.dot_general(sp, wsp_ref[...], (((1,), (0,)), ((), ())),
                                 preferred_element_type=jnp.float32
                                 ) + bsp_ref[...]
        res = so[:, 0:12]
        bl = so[:, 12:16]
        mx = jnp.max(bl, axis=1, keepdims=True)
        e = jnp.exp(bl - mx)
        sb = e / jnp.sum(e, axis=1, keepdims=True)
        sb = jnp.where(jnp.isfinite(sb), sb, 0.0)
        res = jnp.where(jnp.isfinite(res), res, 0.0)
        mb = (f2b == jax.lax.broadcasted_iota(jnp.int32, (BT, B), 1)
              ).astype(jnp.float32)
        bins80 = jnp.dot(mb, sb, preferred_element_type=jnp.float32)
        res80 = jnp.dot(mb, res, preferred_element_type=jnp.float32)
        bmx = jnp.max(bins80, axis=1, keepdims=True)
        i4 = jax.lax.broadcasted_iota(jnp.int32, (BT, NBINS), 1)
        win = jnp.min(jnp.where(bins80 >= bmx, i4, NBINS), axis=1,
                      keepdims=True)
        anchor = jnp.zeros((BT, 3), jnp.float32)
        for k in range(NBINS):
            anchor = anchor + jnp.where(win == k, 1.0, 0.0) \
                * res80[:, 3 * k:3 * k + 3]
        centers = cs_ref[:, 0:3] / cs_ref[:, 3:4]
        vel = out5[:, 0:3]
        yaw = out5[:, 3:5]
        out_ref[...] = jnp.concatenate(
            [centers, anchor, yaw, vel, jnp.zeros((BT, 3), jnp.float32),
             bins80], axis=1)


_GM = np.kron(np.eye(NG, dtype=np.float32), np.ones((F // NG, F // NG), np.float32))


def kernel(x, point_cloud, point2frameidx, frame2batchidx, W1, b1, g1, be1,
           W2, b2, g2, be2, Wvw, bvw, Wvo, bvo, Wyaw, byaw, Wvel, bvel,
           Wbin, bbin, Wres, bres):
    f32 = jnp.float32
    bf16 = jnp.bfloat16
    gm = jnp.asarray(_GM)
    ids = point2frameidx.astype(jnp.int32)
    ids_col = ids.reshape(N, 1)
    idsr = ids.reshape(NGRID, NBLK)
    bounds = jnp.stack([idsr[:, 0], idsr[:, -1]], axis=1)

    h1, st1 = pl.pallas_call(
        _stage1_kernel,
        grid=(NGRID,),
        in_specs=[pl.BlockSpec((F, NBLK), lambda i: (0, i)),
                  pl.BlockSpec((F, F), lambda i: (0, 0)),
                  pl.BlockSpec((1, F), lambda i: (0, 0))],
        out_specs=[pl.BlockSpec((NBLK, F), lambda i: (i, 0)),
                   pl.BlockSpec((8, F), lambda i: (0, 0))],
        out_shape=[jax.ShapeDtypeStruct((N, F), bf16),
                   jax.ShapeDtypeStruct((8, F), f32)],
    )(x, W1.T, b1.reshape(1, F))

    h2, st2 = pl.pallas_call(
        _stage2_kernel,
        grid=(NGRID,),
        in_specs=[pl.BlockSpec((NBLK, F), lambda i: (i, 0)),
                  pl.BlockSpec((8, F), lambda i: (0, 0)),
                  pl.BlockSpec((1, F), lambda i: (0, 0)),
                  pl.BlockSpec((1, F), lambda i: (0, 0)),
                  pl.BlockSpec((F, F), lambda i: (0, 0)),
                  pl.BlockSpec((F, F), lambda i: (0, 0)),
                  pl.BlockSpec((1, F), lambda i: (0, 0))],
        out_specs=[pl.BlockSpec((NBLK, F), lambda i: (i, 0)),
                   pl.BlockSpec((8, F), lambda i: (0, 0))],
        out_shape=[jax.ShapeDtypeStruct((N, F), bf16),
                   jax.ShapeDtypeStruct((8, F), f32)],
    )(h1, st1, g1.reshape(1, F), be1.reshape(1, F), gm, W2.T, b2.reshape(1, F))

    wv8 = jnp.concatenate([Wvw, Wvo, jnp.zeros((4, F), f32)], axis=0).T
    bv8 = jnp.concatenate([bvw, bvo, jnp.zeros((4,), f32)]).reshape(1, 8)
    w5 = jnp.concatenate([Wvel, Wyaw, jnp.zeros((3, F), f32)], axis=0).T
    b5 = jnp.concatenate([bvel, byaw, jnp.zeros((3,), f32)]).reshape(1, 8)
    wsp = jnp.concatenate([Wres, Wbin], axis=0).T
    bsp = jnp.concatenate([bres, bbin]).reshape(1, 16)
    f2b = frame2batchidx.astype(jnp.int32).reshape(BT, 1)

    out = pl.pallas_call(
        _stage3_kernel,
        grid_spec=pltpu.PrefetchScalarGridSpec(
            num_scalar_prefetch=1,
            grid=(NGRID,),
            in_specs=[pl.BlockSpec((NBLK, F), lambda i, b: (i, 0)),
                      pl.BlockSpec((NBLK, 1), lambda i, b: (i, 0)),
                      pl.BlockSpec((NBLK, 4), lambda i, b: (i, 0)),
                      pl.BlockSpec((8, F), lambda i, b: (0, 0)),
                      pl.BlockSpec((1, F), lambda i, b: (0, 0)),
                      pl.BlockSpec((1, F), lambda i, b: (0, 0)),
                      pl.BlockSpec((F, F), lambda i, b: (0, 0)),
                      pl.BlockSpec((F, 8), lambda i, b: (0, 0)),
                      pl.BlockSpec((1, 8), lambda i, b: (0, 0)),
                      pl.BlockSpec((BT, 1), lambda i, b: (0, 0)),
                      pl.BlockSpec((F, 8), lambda i, b: (0, 0)),
                      pl.BlockSpec((1, 8), lambda i, b: (0, 0)),
                      pl.BlockSpec((F, 16), lambda i, b: (0, 0)),
                      pl.BlockSpec((1, 16), lambda i, b: (0, 0))],
            out_specs=pl.BlockSpec((BT, 18), lambda i, b: (0, 0)),
            scratch_shapes=[pltpu.VMEM((BT, F), f32),
                            pltpu.VMEM((BT, 8), f32)],
        ),
        out_shape=jax.ShapeDtypeStruct((BT, 18), f32),
    )(bounds, h2, ids_col, point_cloud, st2, g2.reshape(1, F),
      be2.reshape(1, F), gm, wv8, bv8, f2b, w5, b5, wsp, bsp)
    return out


# speedup vs baseline: 4.7876x; 1.4068x over previous
import jax
import jax.numpy as jnp
import numpy as np
from jax.experimental import pallas as pl
from jax.experimental.pallas import tpu as pltpu

F = 128
N = 262144
BT = 80
B = 8
NBINS = 4
NG = 16
NBLK = 4096
NGRID = N // NBLK
EPS = 1e-5
CNT = float((F // NG) * N)
FN = float(N)


def _stage1_kernel(x_ref, w_ref, b_ref, h_ref, st_ref):
    i = pl.program_id(0)

    @pl.when(i == 0)
    def _():
        st_ref[...] = jnp.zeros_like(st_ref)

    h = jax.lax.dot_general(x_ref[...], w_ref[...], (((0,), (0,)), ((), ())),
                            preferred_element_type=jnp.float32) + b_ref[...]
    h_ref[...] = h.astype(jnp.bfloat16)
    ones = jnp.ones((1, NBLK), jnp.float32)
    st_ref[0:1, :] += jax.lax.dot_general(
        ones, h, (((1,), (0,)), ((), ())), preferred_element_type=jnp.float32)
    st_ref[1:2, :] += jax.lax.dot_general(
        ones, h * h, (((1,), (0,)), ((), ())),
        preferred_element_type=jnp.float32)


def _affine(s, ss, g, be, gm):
    sg = jnp.dot(s, gm, preferred_element_type=jnp.float32)
    sgg = jnp.dot(ss, gm, preferred_element_type=jnp.float32)
    mu = sg / CNT
    var = sgg / CNT - mu * mu
    a = g * jax.lax.rsqrt(var + EPS)
    d = be - mu * a
    return a, d


def _stage2_kernel(h1_ref, st_ref, g_ref, be_ref, gm_ref, gram_ref, s2_ref):
    i = pl.program_id(0)

    @pl.when(i == 0)
    def _():
        gram_ref[...] = jnp.zeros_like(gram_ref)
        s2_ref[...] = jnp.zeros_like(s2_ref)

    a, d = _affine(st_ref[0:1, :], st_ref[1:2, :], g_ref[...], be_ref[...],
                   gm_ref[...])
    h1r = jnp.maximum(h1_ref[...].astype(jnp.float32) * a + d, 0.0)
    gram_ref[...] += jax.lax.dot_general(
        h1r, h1r, (((0,), (0,)), ((), ())),
        preferred_element_type=jnp.float32)
    s2_ref[0:1, :] += jax.lax.dot_general(
        jnp.ones((1, NBLK), jnp.float32), h1r, (((1,), (0,)), ((), ())),
        preferred_element_type=jnp.float32)


def _stage3_kernel(bounds_ref, h1_ref, ids_ref, idsr_ref, pc_ref, st_ref,
                   g1_ref, be1_ref, g2_ref, be2_ref, gm_ref, w2_ref, b2_ref,
                   gram_ref, s2_ref, wv_ref, bv_ref, f2b_ref, w5_ref, b5_ref,
                   wsp_ref, bsp_ref, out_ref, fp_ref, cs_ref):
    i = pl.program_id(0)

    @pl.when(i == 0)
    def _():
        fp_ref[...] = jnp.full_like(fp_ref, -jnp.inf)
        cs_ref[...] = jnp.zeros_like(cs_ref)

    a1, d1 = _affine(st_ref[0:1, :], st_ref[1:2, :], g1_ref[...],
                     be1_ref[...], gm_ref[...])
    h1r = jnp.maximum(h1_ref[...].astype(jnp.float32) * a1 + d1, 0.0)
    b2 = b2_ref[...]
    h2 = jax.lax.dot_general(h1r, w2_ref[...], (((1,), (0,)), ((), ())),
                             preferred_element_type=jnp.float32) + b2

    u = jnp.dot(s2_ref[0:1, :], w2_ref[...],
                preferred_element_type=jnp.float32)
    aw = jnp.dot(gram_ref[...], w2_ref[...],
                 preferred_element_type=jnp.float32)
    diag = jnp.sum(w2_ref[...] * aw, axis=0, keepdims=True)
    s_h2 = u + FN * b2
    ss_h2 = diag + 2.0 * b2 * u + FN * b2 * b2
    a2, d2 = _affine(s_h2, ss_h2, g2_ref[...], be2_ref[...], gm_ref[...])

    xt = jnp.maximum(h2 * a2 + d2, 0.0)
    votes = jax.lax.dot_general(wv_ref[...], xt, (((1,), (1,)), ((), ())),
                                preferred_element_type=jnp.float32
                                ) + bv_ref[...]
    vw = jnp.maximum(jax.nn.sigmoid(votes[0:1, :]), 1e-5)
    cv = (pc_ref[0:3, :] + votes[1:4, :]) * vw
    cvw = jnp.concatenate(
        [cv, vw, jnp.zeros((4, NBLK), jnp.float32)], axis=0)
    ids = ids_ref[...]
    ids_row = idsr_ref[0]
    mt = (ids_row == jax.lax.broadcasted_iota(jnp.int32, (BT, NBLK), 0)
          ).astype(jnp.float32)
    cs_ref[...] += jax.lax.dot_general(mt, cvw, (((1,), (1,)), ((), ())),
                                       preferred_element_type=jnp.float32)

    f0 = bounds_ref[i, 0]
    f1 = bounds_ref[i, 1]

    def body(f, carry):
        m = (ids == f)
        loc = jnp.max(jnp.where(m, xt, -jnp.inf), axis=0, keepdims=True)
        fp_ref[pl.ds(f, 1), :] = jnp.maximum(fp_ref[pl.ds(f, 1), :], loc)
        return carry

    jax.lax.fori_loop(f0, f1 + 1, body, 0)

    @pl.when(i == NGRID - 1)
    def _():
        fp = fp_ref[...]
        f2b = f2b_ref[...]
        rows = []
        for b in range(B):
            m = (f2b == b)
            rows.append(jnp.max(jnp.where(m, fp, -jnp.inf), axis=0,
                                keepdims=True))
        sp = jnp.concatenate(rows, axis=0)
        out5 = jax.lax.dot_general(fp, w5_ref[...], (((1,), (0,)), ((), ())),
                                   preferred_element_type=jnp.float32
                                   ) + b5_ref[...]
        so = jax.lax.dot_general(sp, wsp_ref[...], (((1,), (0,)), ((), ())),
                                 preferred_element_type=jnp.float32
                                 ) + bsp_ref[...]
        res = so[:, 0:12]
        bl = so[:, 12:16]
        mx = jnp.max(bl, axis=1, keepdims=True)
        e = jnp.exp(bl - mx)
        sb = e / jnp.sum(e, axis=1, keepdims=True)
        sb = jnp.where(jnp.isfinite(sb), sb, 0.0)
        res = jnp.where(jnp.isfinite(res), res, 0.0)
        mb = (f2b == jax.lax.broadcasted_iota(jnp.int32, (BT, B), 1)
              ).astype(jnp.float32)
        bins80 = jnp.dot(mb, sb, preferred_element_type=jnp.float32)
        res80 = jnp.dot(mb, res, preferred_element_type=jnp.float32)
        bmx = jnp.max(bins80, axis=1, keepdims=True)
        i4 = jax.lax.broadcasted_iota(jnp.int32, (BT, NBINS), 1)
        win = jnp.min(jnp.where(bins80 >= bmx, i4, NBINS), axis=1,
                      keepdims=True)
        anchor = jnp.zeros((BT, 3), jnp.float32)
        for k in range(NBINS):
            anchor = anchor + jnp.where(win == k, 1.0, 0.0) \
                * res80[:, 3 * k:3 * k + 3]
        centers = cs_ref[:, 0:3] / cs_ref[:, 3:4]
        vel = out5[:, 0:3]
        yaw = out5[:, 3:5]
        out_ref[...] = jnp.concatenate(
            [centers, anchor, yaw, vel, jnp.zeros((BT, 3), jnp.float32),
             bins80], axis=1)


_GM = np.kron(np.eye(NG, dtype=np.float32), np.ones((F // NG, F // NG), np.float32))


def kernel(x, point_cloud, point2frameidx, frame2batchidx, W1, b1, g1, be1,
           W2, b2, g2, be2, Wvw, bvw, Wvo, bvo, Wyaw, byaw, Wvel, bvel,
           Wbin, bbin, Wres, bres):
    f32 = jnp.float32
    bf16 = jnp.bfloat16
    gm = jnp.asarray(_GM)
    ids = point2frameidx.astype(jnp.int32)
    ids_col = ids.reshape(N, 1)
    idsr = ids.reshape(NGRID, NBLK)
    bounds = jnp.stack([idsr[:, 0], idsr[:, -1]], axis=1)

    h1, st1 = pl.pallas_call(
        _stage1_kernel,
        grid=(NGRID,),
        in_specs=[pl.BlockSpec((F, NBLK), lambda i: (0, i)),
                  pl.BlockSpec((F, F), lambda i: (0, 0)),
                  pl.BlockSpec((1, F), lambda i: (0, 0))],
        out_specs=[pl.BlockSpec((NBLK, F), lambda i: (i, 0)),
                   pl.BlockSpec((8, F), lambda i: (0, 0))],
        out_shape=[jax.ShapeDtypeStruct((N, F), bf16),
                   jax.ShapeDtypeStruct((8, F), f32)],
    )(x, W1.T, b1.reshape(1, F))

    gram, s2 = pl.pallas_call(
        _stage2_kernel,
        grid=(NGRID,),
        in_specs=[pl.BlockSpec((NBLK, F), lambda i: (i, 0)),
                  pl.BlockSpec((8, F), lambda i: (0, 0)),
                  pl.BlockSpec((1, F), lambda i: (0, 0)),
                  pl.BlockSpec((1, F), lambda i: (0, 0)),
                  pl.BlockSpec((F, F), lambda i: (0, 0))],
        out_specs=[pl.BlockSpec((F, F), lambda i: (0, 0)),
                   pl.BlockSpec((8, F), lambda i: (0, 0))],
        out_shape=[jax.ShapeDtypeStruct((F, F), f32),
                   jax.ShapeDtypeStruct((8, F), f32)],
    )(h1, st1, g1.reshape(1, F), be1.reshape(1, F), gm)

    wv8 = jnp.concatenate([Wvw, Wvo, jnp.zeros((4, F), f32)], axis=0)
    bv8 = jnp.concatenate([bvw, bvo, jnp.zeros((4,), f32)]).reshape(8, 1)
    pc8 = jnp.concatenate([point_cloud.T[:3], jnp.zeros((5, N), f32)], axis=0)
    ids3 = ids.reshape(NGRID, 1, NBLK)
    w5 = jnp.concatenate([Wvel, Wyaw, jnp.zeros((3, F), f32)], axis=0).T
    b5 = jnp.concatenate([bvel, byaw, jnp.zeros((3,), f32)]).reshape(1, 8)
    wsp = jnp.concatenate([Wres, Wbin], axis=0).T
    bsp = jnp.concatenate([bres, bbin]).reshape(1, 16)
    f2b = frame2batchidx.astype(jnp.int32).reshape(BT, 1)

    out = pl.pallas_call(
        _stage3_kernel,
        grid_spec=pltpu.PrefetchScalarGridSpec(
            num_scalar_prefetch=1,
            grid=(NGRID,),
            in_specs=[pl.BlockSpec((NBLK, F), lambda i, b: (i, 0)),
                      pl.BlockSpec((NBLK, 1), lambda i, b: (i, 0)),
                      pl.BlockSpec((1, 1, NBLK), lambda i, b: (i, 0, 0)),
                      pl.BlockSpec((8, NBLK), lambda i, b: (0, i)),
                      pl.BlockSpec((8, F), lambda i, b: (0, 0)),
                      pl.BlockSpec((1, F), lambda i, b: (0, 0)),
                      pl.BlockSpec((1, F), lambda i, b: (0, 0)),
                      pl.BlockSpec((1, F), lambda i, b: (0, 0)),
                      pl.BlockSpec((1, F), lambda i, b: (0, 0)),
                      pl.BlockSpec((F, F), lambda i, b: (0, 0)),
                      pl.BlockSpec((F, F), lambda i, b: (0, 0)),
                      pl.BlockSpec((1, F), lambda i, b: (0, 0)),
                      pl.BlockSpec((F, F), lambda i, b: (0, 0)),
                      pl.BlockSpec((8, F), lambda i, b: (0, 0)),
                      pl.BlockSpec((8, F), lambda i, b: (0, 0)),
                      pl.BlockSpec((8, 1), lambda i, b: (0, 0)),
                      pl.BlockSpec((BT, 1), lambda i, b: (0, 0)),
                      pl.BlockSpec((F, 8), lambda i, b: (0, 0)),
                      pl.BlockSpec((1, 8), lambda i, b: (0, 0)),
                      pl.BlockSpec((F, 16), lambda i, b: (0, 0)),
                      pl.BlockSpec((1, 16), lambda i, b: (0, 0))],
            out_specs=pl.BlockSpec((BT, 18), lambda i, b: (0, 0)),
            scratch_shapes=[pltpu.VMEM((BT, F), f32),
                            pltpu.VMEM((BT, 8), f32)],
        ),
        out_shape=jax.ShapeDtypeStruct((BT, 18), f32),
    )(bounds, h1, ids_col, ids3, pc8, st1, g1.reshape(1, F),
      be1.reshape(1, F), g2.reshape(1, F), be2.reshape(1, F), gm, W2.T,
      b2.reshape(1, F), gram, s2, wv8, bv8, f2b, w5, b5, wsp, bsp)
    return out
